# Initial kernel scaffold; baseline (speedup 1.0000x reference)
import jax
import jax.numpy as jnp
from jax.experimental import pallas as pl


def kernel(x, edge_index, W1, b1, W2, b2):
    raise NotImplementedError("write your pallas kernel here")


# speedup vs baseline: 2.0026x; 2.0026x over previous
import jax
import jax.numpy as jnp
from jax.experimental import pallas as pl
from jax.experimental.pallas import tpu as pltpu

_LAMB = 1.0


def _finish_body(u_ref, w_ref, b_ref, o_ref):
    u = u_ref[...]
    v = u @ w_ref[...] + b_ref[...][None, :]
    m = jnp.max(v, axis=-1, keepdims=True)
    e = jnp.exp(v - m)
    o_ref[...] = v - m - jnp.log(jnp.sum(e, axis=-1, keepdims=True))


def _finish(u, W2, b2):
    N, D = u.shape
    BR = 1000
    return pl.pallas_call(
        _finish_body,
        grid=(N // BR,),
        in_specs=[
            pl.BlockSpec((BR, D), lambda i: (i, 0)),
            pl.BlockSpec((D, D), lambda i: (0, 0)),
            pl.BlockSpec((D,), lambda i: (0,)),
        ],
        out_specs=pl.BlockSpec((BR, D), lambda i: (i, 0)),
        out_shape=jax.ShapeDtypeStruct((N, D), jnp.float32),
    )(u, W2, b2)


def _gcn_agg(z, row, col, dn, cvec, x_orig):
    N = x_orig.shape[0]
    msg = jnp.take(z, col, axis=0)
    out = jnp.zeros((N, z.shape[1]), jnp.float32).at[row].add(msg)
    return out + cvec[:, None] * x_orig


def kernel(x, edge_index, W1, b1, W2, b2):
    N = x.shape[0]
    row = edge_index[0].astype(jnp.int32)
    col = edge_index[1].astype(jnp.int32)
    deg = jnp.ones((N,), jnp.float32).at[col].add(1.0)
    dn = 1.0 / deg
    diagA = jnp.ones((N,), jnp.float32).at[row].add((row == col).astype(jnp.float32))
    cvec = dn + _LAMB * diagA
    u1 = _gcn_agg(dn[:, None] * x, row, col, dn, cvec, x)
    h = jax.nn.relu(u1 @ W1 + b1)
    u2 = _gcn_agg(dn[:, None] * h, row, col, dn, cvec, h)
    return _finish(u2, W2, b2)


# speedup vs baseline: 6.2399x; 3.1159x over previous
import jax
import jax.numpy as jnp
from jax import lax
from jax.experimental import pallas as pl
from jax.experimental.pallas import tpu as pltpu
from jax.experimental.pallas import tpu_sc as plsc

_LAMB = 1.0
_N = 10000
_E = 320000
_D = 128
_NC = 2
_NS = 16
_NW = _NC * _NS
_EPW = _E // _NW
_K = 80
_NCHUNK = _EPW // _K
_NP = 10240
_RPT = _NP // _NS


def _agg_body(z_hbm, row_hbm, col_hbm, zeros_hbm, parts_hbm,
              idx_row, idx_col, buf, acc, sem):
    cid = lax.axis_index("c")
    sid = lax.axis_index("s")
    wid = cid * _NS + sid
    base = wid * _EPW
    pltpu.sync_copy(zeros_hbm.at[pl.ds(sid * _RPT, _RPT)],
                    acc.at[pl.ds(sid * _RPT, _RPT)])
    plsc.subcore_barrier()

    def step(j, carry):
        off = base + j * _K
        pltpu.sync_copy(col_hbm.at[pl.ds(off, _K)], idx_col)
        pltpu.sync_copy(row_hbm.at[pl.ds(off, _K)], idx_row)
        pltpu.async_copy(z_hbm.at[idx_col], buf, sem).wait()
        pltpu.sync_copy(buf, acc.at[idx_row], add=True)
        return carry

    lax.fori_loop(0, _NCHUNK, step, 0)
    plsc.subcore_barrier()
    pltpu.sync_copy(acc.at[pl.ds(sid * _RPT, _RPT)],
                    parts_hbm.at[cid, pl.ds(sid * _RPT, _RPT)])


_agg = pl.kernel(
    _agg_body,
    out_type=jax.ShapeDtypeStruct((_NC, _NP, _D), jnp.float32),
    mesh=plsc.VectorSubcoreMesh(core_axis_name="c", subcore_axis_name="s"),
    scratch_types=[
        pltpu.VMEM((_K,), jnp.int32),
        pltpu.VMEM((_K,), jnp.int32),
        pltpu.VMEM((_K, _D), jnp.float32),
        pltpu.VMEM_SHARED((_NP, _D), jnp.float32),
        pltpu.SemaphoreType.DMA,
    ],
)


def _finish_body(u_ref, w_ref, b_ref, o_ref):
    u = u_ref[...]
    v = u @ w_ref[...] + b_ref[...][None, :]
    m = jnp.max(v, axis=-1, keepdims=True)
    e = jnp.exp(v - m)
    o_ref[...] = v - m - jnp.log(jnp.sum(e, axis=-1, keepdims=True))


def _finish(u, W2, b2):
    BR = 1000
    return pl.pallas_call(
        _finish_body,
        grid=(_N // BR,),
        in_specs=[
            pl.BlockSpec((BR, _D), lambda i: (i, 0)),
            pl.BlockSpec((_D, _D), lambda i: (0, 0)),
            pl.BlockSpec((_D,), lambda i: (0,)),
        ],
        out_specs=pl.BlockSpec((BR, _D), lambda i: (i, 0)),
        out_shape=jax.ShapeDtypeStruct((_N, _D), jnp.float32),
    )(u, W2, b2)


def kernel(x, edge_index, W1, b1, W2, b2):
    row = edge_index[0].astype(jnp.int32)
    col = edge_index[1].astype(jnp.int32)
    deg = jnp.ones((_N,), jnp.float32).at[col].add(1.0)
    dn = 1.0 / deg
    diagA = jnp.ones((_N,), jnp.float32).at[row].add((row == col).astype(jnp.float32))
    cvec = dn + _LAMB * diagA
    zeros = jnp.zeros((_NP, _D), jnp.float32)

    p1 = _agg(dn[:, None] * x, row, col, zeros)
    u1 = p1[0, :_N] + p1[1, :_N] + cvec[:, None] * x
    h = jax.nn.relu(u1 @ W1 + b1)
    p2 = _agg(dn[:, None] * h, row, col, zeros)
    u2 = p2[0, :_N] + p2[1, :_N] + cvec[:, None] * h
    return _finish(u2, W2, b2)


# speedup vs baseline: 15.9165x; 2.5508x over previous
import jax
import jax.numpy as jnp
from jax import lax
from jax.experimental import pallas as pl
from jax.experimental.pallas import tpu as pltpu
from jax.experimental.pallas import tpu_sc as plsc

_LAMB = 1.0
_N = 10000
_E = 320000
_D = 128
_NC = 2
_NS = 16
_NW = _NC * _NS
_EPW = _E // _NW
_K = 80
_NCHUNK = _EPW // _K
_NP = 10240
_RPT = _NP // _NS


def _deg_body(row_hbm, col_hbm, degp_hbm, diagp_hbm, rowv, colv, hist, hist2):
    cid = lax.axis_index("c")
    sid = lax.axis_index("s")
    wid = cid * _NS + sid
    pltpu.sync_copy(row_hbm.at[wid], rowv)
    pltpu.sync_copy(col_hbm.at[wid], colv)
    zeros16 = jnp.zeros((16,), jnp.float32)

    def zloop(i, c):
        hist[pl.ds(i * 16, 16)] = zeros16
        hist2[pl.ds(i * 16, 16)] = zeros16
        return c

    lax.fori_loop(0, _N // 16, zloop, 0)
    ones16 = jnp.ones((16,), jnp.float32)

    def step(i, c):
        c16 = colv[pl.ds(i * 16, 16)]
        plsc.addupdate_scatter(hist, [c16], ones16)
        r16 = rowv[pl.ds(i * 16, 16)]
        plsc.addupdate_scatter(hist2, [r16], ones16, mask=r16 == c16)
        return c

    lax.fori_loop(0, _EPW // 16, step, 0)
    pltpu.sync_copy(hist, degp_hbm.at[wid])
    pltpu.sync_copy(hist2, diagp_hbm.at[wid])


_deg = pl.kernel(
    _deg_body,
    out_type=(
        jax.ShapeDtypeStruct((_NW, _N), jnp.float32),
        jax.ShapeDtypeStruct((_NW, _N), jnp.float32),
    ),
    mesh=plsc.VectorSubcoreMesh(core_axis_name="c", subcore_axis_name="s"),
    compiler_params=pltpu.CompilerParams(needs_layout_passes=False),
    scratch_types=[
        pltpu.VMEM((_EPW,), jnp.int32),
        pltpu.VMEM((_EPW,), jnp.int32),
        pltpu.VMEM((_N,), jnp.float32),
        pltpu.VMEM((_N,), jnp.float32),
    ],
)


def _agg_body(z_hbm, row_hbm, col_hbm, zeros_hbm, parts_hbm,
              rowv, colv, buf, acc, gsem, ssem):
    cid = lax.axis_index("c")
    sid = lax.axis_index("s")
    wid = cid * _NS + sid
    pltpu.sync_copy(zeros_hbm.at[pl.ds(sid * _RPT, _RPT)],
                    acc.at[pl.ds(sid * _RPT, _RPT)])
    pltpu.sync_copy(row_hbm.at[wid], rowv)
    pltpu.sync_copy(col_hbm.at[wid], colv)
    plsc.subcore_barrier()

    def step(j, carry):
        pltpu.async_copy(z_hbm.at[colv.at[j]], buf, gsem).wait()
        pltpu.async_copy(buf, acc.at[rowv.at[j]], ssem, add=True).wait()
        return carry

    lax.fori_loop(0, _NCHUNK, step, 0)
    plsc.subcore_barrier()
    pltpu.sync_copy(acc.at[pl.ds(sid * _RPT, _RPT)],
                    parts_hbm.at[cid, pl.ds(sid * _RPT, _RPT)])


_agg = pl.kernel(
    _agg_body,
    out_type=jax.ShapeDtypeStruct((_NC, _NP, _D), jnp.float32),
    mesh=plsc.VectorSubcoreMesh(core_axis_name="c", subcore_axis_name="s"),
    scratch_types=[
        pltpu.VMEM((_NCHUNK, _K), jnp.int32),
        pltpu.VMEM((_NCHUNK, _K), jnp.int32),
        pltpu.VMEM((_K, _D), jnp.float32),
        pltpu.VMEM_SHARED((_NP, _D), jnp.float32),
        pltpu.SemaphoreType.DMA,
        pltpu.SemaphoreType.DMA,
    ],
)


def _prep_body(degp_ref, diagp_ref, x_ref, z_ref, dn_ref, cv_ref):
    deg = 1.0 + jnp.sum(degp_ref[...], axis=0)
    dn = 1.0 / deg
    cv = dn + _LAMB * (1.0 + jnp.sum(diagp_ref[...], axis=0))
    dn_ref[...] = dn[:, None]
    cv_ref[...] = cv[:, None]
    z_ref[...] = dn[:, None] * x_ref[...]


def _prep(degp, diagp, x):
    return pl.pallas_call(
        _prep_body,
        out_shape=(
            jax.ShapeDtypeStruct((_N, _D), jnp.float32),
            jax.ShapeDtypeStruct((_N, 1), jnp.float32),
            jax.ShapeDtypeStruct((_N, 1), jnp.float32),
        ),
    )(degp, diagp, x)


def _layer1_body(p_ref, x_ref, cv_ref, dn_ref, w_ref, b_ref, y_ref, z_ref):
    u = p_ref[0] + p_ref[1] + cv_ref[...] * x_ref[...]
    y = jnp.maximum(u @ w_ref[...] + b_ref[...][None, :], 0.0)
    y_ref[...] = y
    z_ref[...] = dn_ref[...] * y


def _layer1(p, x, cv, dn, W1, b1):
    BR = 1000
    return pl.pallas_call(
        _layer1_body,
        grid=(_N // BR,),
        in_specs=[
            pl.BlockSpec((_NC, BR, _D), lambda i: (0, i, 0)),
            pl.BlockSpec((BR, _D), lambda i: (i, 0)),
            pl.BlockSpec((BR, 1), lambda i: (i, 0)),
            pl.BlockSpec((BR, 1), lambda i: (i, 0)),
            pl.BlockSpec((_D, _D), lambda i: (0, 0)),
            pl.BlockSpec((_D,), lambda i: (0,)),
        ],
        out_specs=(
            pl.BlockSpec((BR, _D), lambda i: (i, 0)),
            pl.BlockSpec((BR, _D), lambda i: (i, 0)),
        ),
        out_shape=(
            jax.ShapeDtypeStruct((_N, _D), jnp.float32),
            jax.ShapeDtypeStruct((_N, _D), jnp.float32),
        ),
    )(p, x, cv, dn, W1, b1)


def _layer2_body(p_ref, y_ref, cv_ref, w_ref, b_ref, o_ref):
    u = p_ref[0] + p_ref[1] + cv_ref[...] * y_ref[...]
    v = u @ w_ref[...] + b_ref[...][None, :]
    m = jnp.max(v, axis=-1, keepdims=True)
    e = jnp.exp(v - m)
    o_ref[...] = v - m - jnp.log(jnp.sum(e, axis=-1, keepdims=True))


def _layer2(p, y1, cv, W2, b2):
    BR = 1000
    return pl.pallas_call(
        _layer2_body,
        grid=(_N // BR,),
        in_specs=[
            pl.BlockSpec((_NC, BR, _D), lambda i: (0, i, 0)),
            pl.BlockSpec((BR, _D), lambda i: (i, 0)),
            pl.BlockSpec((BR, 1), lambda i: (i, 0)),
            pl.BlockSpec((_D, _D), lambda i: (0, 0)),
            pl.BlockSpec((_D,), lambda i: (0,)),
        ],
        out_specs=pl.BlockSpec((BR, _D), lambda i: (i, 0)),
        out_shape=jax.ShapeDtypeStruct((_N, _D), jnp.float32),
    )(p, y1, cv, W2, b2)


def kernel(x, edge_index, W1, b1, W2, b2):
    row = edge_index[0].astype(jnp.int32)
    col = edge_index[1].astype(jnp.int32)
    rowf = row.reshape(_NW, _EPW)
    colf = col.reshape(_NW, _EPW)
    row3 = row.reshape(_NW, _NCHUNK, _K)
    col3 = col.reshape(_NW, _NCHUNK, _K)
    zeros = jnp.zeros((_NP, _D), jnp.float32)

    degp, diagp = _deg(rowf, colf)
    z1, dn, cv = _prep(degp, diagp, x)
    p1 = _agg(z1, row3, col3, zeros)
    y1, z2 = _layer1(p1[:, :_N], x, cv, dn, W1, b1)
    p2 = _agg(z2, row3, col3, zeros)
    return _layer2(p2[:, :_N], y1, cv, W2, b2)


# speedup vs baseline: 26.8172x; 1.6849x over previous
import jax
import jax.numpy as jnp
from jax import lax
from jax.experimental import pallas as pl
from jax.experimental.pallas import tpu as pltpu
from jax.experimental.pallas import tpu_sc as plsc

_LAMB = 1.0
_N = 10000
_E = 320000
_D = 128
_NC = 2
_NS = 16
_NW = _NC * _NS
_EPW = _E // _NW
_K = 80
_NCHUNK = _EPW // _K
_NP = 10240
_RPT = _NP // _NS


def _deg_body(row_hbm, col_hbm, degp_hbm, diagp_hbm, rowv, colv, hist, hist2):
    cid = lax.axis_index("c")
    sid = lax.axis_index("s")
    wid = cid * _NS + sid
    pltpu.sync_copy(row_hbm.at[wid], rowv)
    pltpu.sync_copy(col_hbm.at[wid], colv)
    zeros16 = jnp.zeros((16,), jnp.float32)

    def zloop(i, c):
        hist[pl.ds(i * 16, 16)] = zeros16
        hist2[pl.ds(i * 16, 16)] = zeros16
        return c

    lax.fori_loop(0, _N // 16, zloop, 0)
    ones16 = jnp.ones((16,), jnp.float32)

    def step(i, c):
        c16 = colv[pl.ds(i * 16, 16)]
        plsc.addupdate_scatter(hist, [c16], ones16)
        r16 = rowv[pl.ds(i * 16, 16)]
        plsc.addupdate_scatter(hist2, [r16], ones16, mask=r16 == c16)
        return c

    lax.fori_loop(0, _EPW // 16, step, 0)
    pltpu.sync_copy(hist, degp_hbm.at[wid])
    pltpu.sync_copy(hist2, diagp_hbm.at[wid])


_deg = pl.kernel(
    _deg_body,
    out_type=(
        jax.ShapeDtypeStruct((_NW, _N), jnp.float32),
        jax.ShapeDtypeStruct((_NW, _N), jnp.float32),
    ),
    mesh=plsc.VectorSubcoreMesh(core_axis_name="c", subcore_axis_name="s"),
    compiler_params=pltpu.CompilerParams(needs_layout_passes=False),
    scratch_types=[
        pltpu.VMEM((_EPW,), jnp.int32),
        pltpu.VMEM((_EPW,), jnp.int32),
        pltpu.VMEM((_N,), jnp.float32),
        pltpu.VMEM((_N,), jnp.float32),
    ],
)


_NB = 3


def _agg_body(z_hbm, row_hbm, col_hbm, zeros_hbm, parts_hbm,
              ic0, ic1, ic2, ir0, ir1, ir2, b0, b1, b2, acc,
              gA, gB, gC, sA, sB, sC, icA, icB, icC, irA, irB, irC):
    ics = (ic0, ic1, ic2)
    irs = (ir0, ir1, ir2)
    bufs = (b0, b1, b2)
    gs = (gA, gB, gC)
    ss = (sA, sB, sC)
    icsem = (icA, icB, icC)
    irsem = (irA, irB, irC)
    cid = lax.axis_index("c")
    sid = lax.axis_index("s")
    wid = cid * _NS + sid
    base = wid * _EPW
    pltpu.sync_copy(zeros_hbm.at[pl.ds(sid * _RPT, _RPT)],
                    acc.at[pl.ds(sid * _RPT, _RPT)])
    plsc.subcore_barrier()

    def ic_start(j, p):
        pltpu.async_copy(col_hbm.at[pl.ds(base + j * _K, _K)], ics[p],
                         icsem[p])

    def ic_wait(p):
        pltpu.make_async_copy(col_hbm.at[pl.ds(base, _K)], ics[p],
                              icsem[p]).wait()

    def ir_start(j, p):
        pltpu.async_copy(row_hbm.at[pl.ds(base + j * _K, _K)], irs[p],
                         irsem[p])

    def ir_wait(p):
        pltpu.make_async_copy(row_hbm.at[pl.ds(base, _K)], irs[p],
                              irsem[p]).wait()

    def g_start(p):
        pltpu.async_copy(z_hbm.at[ics[p]], bufs[p], gs[p])

    def g_wait(p):
        pltpu.make_async_copy(z_hbm.at[ics[p]], bufs[p], gs[p]).wait()

    def s_start(p):
        pltpu.async_copy(bufs[p], acc.at[irs[p]], ss[p], add=True)

    def s_wait(p):
        pltpu.make_async_copy(bufs[p], acc.at[irs[p]], ss[p]).wait()

    ic_start(0, 0)
    ic_start(1, 1)
    ic_start(2, 2)
    ic_wait(0)
    g_start(0)
    ic_wait(1)
    g_start(1)
    ir_start(0, 0)
    ir_start(1, 1)
    ng = (_NCHUNK + _NB - 1) // _NB + 1

    def group(i, carry):
        jb = i * _NB
        for b in range(_NB):
            j = jb + b
            pg = (b + 2) % _NB

            @pl.when((j >= 1) & (j <= _NCHUNK))
            def _():
                s_wait(pg)

            ir_start(jnp.minimum(j + 2, _NCHUNK - 1), pg)
            ic_wait(pg)
            g_start(pg)
            g_wait(b)
            ic_start(jnp.minimum(j + 3, _NCHUNK - 1), b)
            ir_wait(b)

            @pl.when(j < _NCHUNK)
            def _():
                s_start(b)

        return carry

    lax.fori_loop(0, ng, group, 0)
    ic_wait(2)
    ir_wait(0)
    ir_wait(1)
    g_wait(0)
    g_wait(1)
    plsc.subcore_barrier()
    pltpu.sync_copy(acc.at[pl.ds(sid * _RPT, _RPT)],
                    parts_hbm.at[cid, pl.ds(sid * _RPT, _RPT)])


_agg = pl.kernel(
    _agg_body,
    out_type=jax.ShapeDtypeStruct((_NC, _NP, _D), jnp.float32),
    mesh=plsc.VectorSubcoreMesh(core_axis_name="c", subcore_axis_name="s"),
    scratch_types=(
        [pltpu.VMEM((_K,), jnp.int32) for _ in range(2 * _NB)]
        + [pltpu.VMEM((_K, _D), jnp.float32) for _ in range(_NB)]
        + [pltpu.VMEM_SHARED((_NP, _D), jnp.float32)]
        + [pltpu.SemaphoreType.DMA for _ in range(4 * _NB)]
    ),
)


def _prep_body(degp_ref, diagp_ref, x_ref, z_ref, dn_ref, cv_ref):
    deg = 1.0 + jnp.sum(degp_ref[...], axis=0)
    dn = 1.0 / deg
    cv = dn + _LAMB * (1.0 + jnp.sum(diagp_ref[...], axis=0))
    dn_ref[...] = dn[:, None]
    cv_ref[...] = cv[:, None]
    z_ref[...] = dn[:, None] * x_ref[...]


def _prep(degp, diagp, x):
    return pl.pallas_call(
        _prep_body,
        out_shape=(
            jax.ShapeDtypeStruct((_N, _D), jnp.float32),
            jax.ShapeDtypeStruct((_N, 1), jnp.float32),
            jax.ShapeDtypeStruct((_N, 1), jnp.float32),
        ),
    )(degp, diagp, x)


def _layer1_body(p_ref, x_ref, cv_ref, dn_ref, w_ref, b_ref, y_ref, z_ref):
    u = p_ref[0] + p_ref[1] + cv_ref[...] * x_ref[...]
    y = jnp.maximum(u @ w_ref[...] + b_ref[...][None, :], 0.0)
    y_ref[...] = y
    z_ref[...] = dn_ref[...] * y


def _layer1(p, x, cv, dn, W1, b1):
    BR = 1000
    return pl.pallas_call(
        _layer1_body,
        grid=(_N // BR,),
        in_specs=[
            pl.BlockSpec((_NC, BR, _D), lambda i: (0, i, 0)),
            pl.BlockSpec((BR, _D), lambda i: (i, 0)),
            pl.BlockSpec((BR, 1), lambda i: (i, 0)),
            pl.BlockSpec((BR, 1), lambda i: (i, 0)),
            pl.BlockSpec((_D, _D), lambda i: (0, 0)),
            pl.BlockSpec((_D,), lambda i: (0,)),
        ],
        out_specs=(
            pl.BlockSpec((BR, _D), lambda i: (i, 0)),
            pl.BlockSpec((BR, _D), lambda i: (i, 0)),
        ),
        out_shape=(
            jax.ShapeDtypeStruct((_N, _D), jnp.float32),
            jax.ShapeDtypeStruct((_N, _D), jnp.float32),
        ),
    )(p, x, cv, dn, W1, b1)


def _layer2_body(p_ref, y_ref, cv_ref, w_ref, b_ref, o_ref):
    u = p_ref[0] + p_ref[1] + cv_ref[...] * y_ref[...]
    v = u @ w_ref[...] + b_ref[...][None, :]
    m = jnp.max(v, axis=-1, keepdims=True)
    e = jnp.exp(v - m)
    o_ref[...] = v - m - jnp.log(jnp.sum(e, axis=-1, keepdims=True))


def _layer2(p, y1, cv, W2, b2):
    BR = 1000
    return pl.pallas_call(
        _layer2_body,
        grid=(_N // BR,),
        in_specs=[
            pl.BlockSpec((_NC, BR, _D), lambda i: (0, i, 0)),
            pl.BlockSpec((BR, _D), lambda i: (i, 0)),
            pl.BlockSpec((BR, 1), lambda i: (i, 0)),
            pl.BlockSpec((_D, _D), lambda i: (0, 0)),
            pl.BlockSpec((_D,), lambda i: (0,)),
        ],
        out_specs=pl.BlockSpec((BR, _D), lambda i: (i, 0)),
        out_shape=jax.ShapeDtypeStruct((_N, _D), jnp.float32),
    )(p, y1, cv, W2, b2)


def kernel(x, edge_index, W1, b1, W2, b2):
    row = edge_index[0].astype(jnp.int32)
    col = edge_index[1].astype(jnp.int32)
    rowf = row.reshape(_NW, _EPW)
    colf = col.reshape(_NW, _EPW)
    zeros = jnp.zeros((_NP, _D), jnp.float32)

    degp, diagp = _deg(rowf, colf)
    z1, dn, cv = _prep(degp, diagp, x)
    p1 = _agg(z1, row, col, zeros)
    y1, z2 = _layer1(p1[:, :_N], x, cv, dn, W1, b1)
    p2 = _agg(z2, row, col, zeros)
    return _layer2(p2[:, :_N], y1, cv, W2, b2)


# speedup vs baseline: 27.7681x; 1.0355x over previous
import jax
import jax.numpy as jnp
from jax import lax
from jax.experimental import pallas as pl
from jax.experimental.pallas import tpu as pltpu
from jax.experimental.pallas import tpu_sc as plsc

_LAMB = 1.0
_N = 10000
_E = 320000
_D = 128
_NC = 2
_NS = 16
_NW = _NC * _NS
_EPW = _E // _NW
_K = 80
_NCHUNK = _EPW // _K
_NP = 10240
_RPT = _NP // _NS


def _deg_body(row_hbm, col_hbm, degp_hbm, diagp_hbm, rowv, colv, hist, hist2):
    cid = lax.axis_index("c")
    sid = lax.axis_index("s")
    wid = cid * _NS + sid
    pltpu.sync_copy(row_hbm.at[wid], rowv)
    pltpu.sync_copy(col_hbm.at[wid], colv)
    zeros16 = jnp.zeros((16,), jnp.float32)

    def zloop(i, c):
        hist[pl.ds(i * 16, 16)] = zeros16
        hist2[pl.ds(i * 16, 16)] = zeros16
        return c

    lax.fori_loop(0, _N // 16, zloop, 0)
    ones16 = jnp.ones((16,), jnp.float32)

    def step(i, c):
        c16 = colv[pl.ds(i * 16, 16)]
        plsc.addupdate_scatter(hist, [c16], ones16)
        r16 = rowv[pl.ds(i * 16, 16)]
        plsc.addupdate_scatter(hist2, [r16], ones16, mask=r16 == c16)
        return c

    lax.fori_loop(0, _EPW // 16, step, 0)
    pltpu.sync_copy(hist, degp_hbm.at[wid])
    pltpu.sync_copy(hist2, diagp_hbm.at[wid])


_deg = pl.kernel(
    _deg_body,
    out_type=(
        jax.ShapeDtypeStruct((_NW, _N), jnp.float32),
        jax.ShapeDtypeStruct((_NW, _N), jnp.float32),
    ),
    mesh=plsc.VectorSubcoreMesh(core_axis_name="c", subcore_axis_name="s"),
    compiler_params=pltpu.CompilerParams(needs_layout_passes=False),
    scratch_types=[
        pltpu.VMEM((_EPW,), jnp.int32),
        pltpu.VMEM((_EPW,), jnp.int32),
        pltpu.VMEM((_N,), jnp.float32),
        pltpu.VMEM((_N,), jnp.float32),
    ],
)


_NB = 4


def _agg_body(z_hbm, row_hbm, col_hbm, zeros_hbm, parts_hbm,
              ic0, ic1, ic2, ic3, ir0, ir1, ir2, ir3, b0, b1, b2, b3, acc,
              gA, gB, gC, gD, sA, sB, sC, sD,
              icA, icB, icC, icD, irA, irB, irC, irD):
    ics = (ic0, ic1, ic2, ic3)
    irs = (ir0, ir1, ir2, ir3)
    bufs = (b0, b1, b2, b3)
    gs = (gA, gB, gC, gD)
    ss = (sA, sB, sC, sD)
    icsem = (icA, icB, icC, icD)
    irsem = (irA, irB, irC, irD)
    cid = lax.axis_index("c")
    sid = lax.axis_index("s")
    wid = cid * _NS + sid
    base = wid * _EPW
    pltpu.sync_copy(zeros_hbm.at[pl.ds(sid * _RPT, _RPT)],
                    acc.at[pl.ds(sid * _RPT, _RPT)])
    plsc.subcore_barrier()

    def ic_start(j, p):
        pltpu.async_copy(col_hbm.at[pl.ds(base + j * _K, _K)], ics[p],
                         icsem[p])

    def ic_wait(p):
        pltpu.make_async_copy(col_hbm.at[pl.ds(base, _K)], ics[p],
                              icsem[p]).wait()

    def ir_start(j, p):
        pltpu.async_copy(row_hbm.at[pl.ds(base + j * _K, _K)], irs[p],
                         irsem[p])

    def ir_wait(p):
        pltpu.make_async_copy(row_hbm.at[pl.ds(base, _K)], irs[p],
                              irsem[p]).wait()

    def g_start(p):
        pltpu.async_copy(z_hbm.at[ics[p]], bufs[p], gs[p])

    def g_wait(p):
        pltpu.make_async_copy(z_hbm.at[ics[p]], bufs[p], gs[p]).wait()

    def s_start(p):
        pltpu.async_copy(bufs[p], acc.at[irs[p]], ss[p], add=True)

    def s_wait(p):
        pltpu.make_async_copy(bufs[p], acc.at[irs[p]], ss[p]).wait()

    ic_start(0, 0)
    ic_start(1, 1)
    ic_start(2, 2)
    ic_start(3, 3)
    ic_wait(0)
    g_start(0)
    ic_wait(1)
    g_start(1)
    ic_wait(2)
    g_start(2)
    ir_start(0, 0)
    ir_start(1, 1)
    ir_start(2, 2)
    ng = (_NCHUNK + 1 + _NB - 1) // _NB

    def group(i, carry):
        jb = i * _NB
        for b in range(_NB):
            j = jb + b
            pg = (b + 3) % _NB

            @pl.when((j >= 1) & (j <= _NCHUNK))
            def _():
                s_wait(pg)

            ir_start(jnp.minimum(j + 3, _NCHUNK - 1), pg)
            ic_wait(pg)
            g_start(pg)
            g_wait(b)
            ic_start(jnp.minimum(j + 4, _NCHUNK - 1), b)
            ir_wait(b)

            @pl.when(j < _NCHUNK)
            def _():
                s_start(b)

        return carry

    lax.fori_loop(0, ng, group, 0)
    ic_wait(3)
    ir_wait(0)
    ir_wait(1)
    ir_wait(2)
    g_wait(0)
    g_wait(1)
    g_wait(2)
    plsc.subcore_barrier()
    pltpu.sync_copy(acc.at[pl.ds(sid * _RPT, _RPT)],
                    parts_hbm.at[cid, pl.ds(sid * _RPT, _RPT)])


_agg = pl.kernel(
    _agg_body,
    out_type=jax.ShapeDtypeStruct((_NC, _NP, _D), jnp.float32),
    mesh=plsc.VectorSubcoreMesh(core_axis_name="c", subcore_axis_name="s"),
    scratch_types=(
        [pltpu.VMEM((_K,), jnp.int32) for _ in range(2 * _NB)]
        + [pltpu.VMEM((_K, _D), jnp.float32) for _ in range(_NB)]
        + [pltpu.VMEM_SHARED((_NP, _D), jnp.float32)]
        + [pltpu.SemaphoreType.DMA for _ in range(4 * _NB)]
    ),
)


def _prep_body(degp_ref, diagp_ref, x_ref, z_ref, dn_ref, cv_ref):
    deg = 1.0 + jnp.sum(degp_ref[...], axis=0)
    dn = 1.0 / deg
    cv = dn + _LAMB * (1.0 + jnp.sum(diagp_ref[...], axis=0))
    dn_ref[...] = dn[:, None]
    cv_ref[...] = cv[:, None]
    z_ref[...] = dn[:, None] * x_ref[...]


def _prep(degp, diagp, x):
    return pl.pallas_call(
        _prep_body,
        out_shape=(
            jax.ShapeDtypeStruct((_N, _D), jnp.float32),
            jax.ShapeDtypeStruct((_N, 1), jnp.float32),
            jax.ShapeDtypeStruct((_N, 1), jnp.float32),
        ),
    )(degp, diagp, x)


def _layer1_body(p_ref, x_ref, cv_ref, dn_ref, w_ref, b_ref, y_ref, z_ref):
    u = p_ref[0] + p_ref[1] + cv_ref[...] * x_ref[...]
    y = jnp.maximum(u @ w_ref[...] + b_ref[...][None, :], 0.0)
    y_ref[...] = y
    z_ref[...] = dn_ref[...] * y


def _layer1(p, x, cv, dn, W1, b1):
    BR = 1000
    return pl.pallas_call(
        _layer1_body,
        grid=(_N // BR,),
        in_specs=[
            pl.BlockSpec((_NC, BR, _D), lambda i: (0, i, 0)),
            pl.BlockSpec((BR, _D), lambda i: (i, 0)),
            pl.BlockSpec((BR, 1), lambda i: (i, 0)),
            pl.BlockSpec((BR, 1), lambda i: (i, 0)),
            pl.BlockSpec((_D, _D), lambda i: (0, 0)),
            pl.BlockSpec((_D,), lambda i: (0,)),
        ],
        out_specs=(
            pl.BlockSpec((BR, _D), lambda i: (i, 0)),
            pl.BlockSpec((BR, _D), lambda i: (i, 0)),
        ),
        out_shape=(
            jax.ShapeDtypeStruct((_N, _D), jnp.float32),
            jax.ShapeDtypeStruct((_N, _D), jnp.float32),
        ),
    )(p, x, cv, dn, W1, b1)


def _layer2_body(p_ref, y_ref, cv_ref, w_ref, b_ref, o_ref):
    u = p_ref[0] + p_ref[1] + cv_ref[...] * y_ref[...]
    v = u @ w_ref[...] + b_ref[...][None, :]
    m = jnp.max(v, axis=-1, keepdims=True)
    e = jnp.exp(v - m)
    o_ref[...] = v - m - jnp.log(jnp.sum(e, axis=-1, keepdims=True))


def _layer2(p, y1, cv, W2, b2):
    BR = 1000
    return pl.pallas_call(
        _layer2_body,
        grid=(_N // BR,),
        in_specs=[
            pl.BlockSpec((_NC, BR, _D), lambda i: (0, i, 0)),
            pl.BlockSpec((BR, _D), lambda i: (i, 0)),
            pl.BlockSpec((BR, 1), lambda i: (i, 0)),
            pl.BlockSpec((_D, _D), lambda i: (0, 0)),
            pl.BlockSpec((_D,), lambda i: (0,)),
        ],
        out_specs=pl.BlockSpec((BR, _D), lambda i: (i, 0)),
        out_shape=jax.ShapeDtypeStruct((_N, _D), jnp.float32),
    )(p, y1, cv, W2, b2)


def kernel(x, edge_index, W1, b1, W2, b2):
    row = edge_index[0].astype(jnp.int32)
    col = edge_index[1].astype(jnp.int32)
    rowf = row.reshape(_NW, _EPW)
    colf = col.reshape(_NW, _EPW)
    zeros = jnp.zeros((_NP, _D), jnp.float32)

    degp, diagp = _deg(rowf, colf)
    z1, dn, cv = _prep(degp, diagp, x)
    p1 = _agg(z1, row, col, zeros)
    y1, z2 = _layer1(p1[:, :_N], x, cv, dn, W1, b1)
    p2 = _agg(z2, row, col, zeros)
    return _layer2(p2[:, :_N], y1, cv, W2, b2)
